# baseline (speedup 1.0000x reference)
import jax
import jax.numpy as jnp
from jax import lax
from jax.experimental import pallas as pl
from jax.experimental.pallas import tpu as pltpu
from jax.experimental.pallas import tpu_sc as plsc

_G = 8
_NC, _NS = 2, 16
_JPW = 6
_R = 128


def _tc_xs_body(node_ref, xs_ref, xs_rep, sem):
    i = pl.program_id(0)
    nsteps = pl.num_programs(0)
    n, _ = node_ref.shape
    slot = jax.lax.rem(i, 2)

    @pl.when(i == 0)
    def _():
        for r in range(_G):
            xs_rep[r * n:(r + 1) * n, :] = node_ref[...]

    @pl.when(i >= 2)
    def _():
        pltpu.make_async_copy(xs_rep, xs_ref.at[i - 2], sem.at[slot]).wait()
    pltpu.make_async_copy(xs_rep, xs_ref.at[i], sem.at[slot]).start()

    @pl.when(i == nsteps - 1)
    def _():
        pltpu.make_async_copy(xs_rep, xs_ref.at[i - 1], sem.at[1 - slot]).wait()
        pltpu.make_async_copy(xs_rep, xs_ref.at[i], sem.at[slot]).wait()


def _sc_xt_body(week_hbm, hour_hbm, minute_hbm, tw_hbm, th_hbm, tm_hbm,
                xt_hbm, idxw, idxh, idxm, wrows, hrows, mrows, rep,
                sem_g, sem_d):
    c = lax.axis_index("c")
    s = lax.axis_index("s")
    wid = s * _NC + c

    pltpu.sync_copy(tw_hbm.at[wid], idxw)
    pltpu.sync_copy(th_hbm.at[wid], idxh)
    pltpu.sync_copy(tm_hbm.at[wid], idxm)
    g1 = pltpu.async_copy(week_hbm.at[idxw], wrows, sem_g)
    g2 = pltpu.async_copy(hour_hbm.at[idxh], hrows, sem_g)
    g3 = pltpu.async_copy(minute_hbm.at[idxm], mrows, sem_g)
    g1.wait()
    g2.wait()
    g3.wait()

    pend = {}
    for j in range(_JPW):
        slot = j % 2
        if j >= 2:
            for h in pend.pop(j - 2):
                h.wait()
        for ch in range(8):
            sl = pl.ds(ch * 16, 16)
            v = wrows[j, sl] + hrows[j, sl] + mrows[j, sl]
            for r in range(_R):
                rep[slot, r, sl] = v
        jj = wid * _JPW + j
        hs = []
        for k in range(512 // _R):
            hs.append(pltpu.async_copy(
                rep.at[slot], xt_hbm.at[jj, pl.ds(k * _R, _R), :],
                sem_d.at[slot]))
        pend[j] = hs
    for j in (_JPW - 2, _JPW - 1):
        for h in pend.pop(j):
            h.wait()


def kernel(t, node_embedding, week_table, hour_table, minute_table):
    B, L = t.shape[0], t.shape[1]
    N, H = node_embedding.shape
    wn, wd = week_table.shape
    hn, hd = hour_table.shape
    mn, md = minute_table.shape
    steps = (B * L) // _G
    nw = _NC * _NS

    week_p = jnp.zeros((8, H), jnp.float32).at[:wn, :wd].set(week_table)
    hour_p = jnp.zeros((24, H), jnp.float32).at[:hn, wd:wd + hd].set(hour_table)
    minute_p = jnp.zeros((8, H), jnp.float32).at[:mn, wd + hd:].set(minute_table)

    def pack(ix):
        return jnp.pad(ix.reshape(nw, _JPW), ((0, 0), (0, 8 - _JPW)))

    tw = pack(t[:, :, 0, 1].reshape(-1).astype(jnp.int32))
    th = pack(t[:, :, 0, 2].reshape(-1).astype(jnp.int32))
    tm = pack(t[:, :, 0, 3].reshape(-1).astype(jnp.int32))

    mesh = plsc.VectorSubcoreMesh(core_axis_name="c", subcore_axis_name="s",
                                  num_cores=_NC, num_subcores=_NS)
    xt = pl.kernel(
        _sc_xt_body,
        out_type=jax.ShapeDtypeStruct((B * L, N, H), jnp.float32),
        mesh=mesh,
        scratch_types=[
            pltpu.VMEM((8,), jnp.int32),
            pltpu.VMEM((8,), jnp.int32),
            pltpu.VMEM((8,), jnp.int32),
            pltpu.VMEM((8, H), jnp.float32),
            pltpu.VMEM((8, H), jnp.float32),
            pltpu.VMEM((8, H), jnp.float32),
            pltpu.VMEM((2, _R, H), jnp.float32),
            pltpu.SemaphoreType.DMA,
            pltpu.SemaphoreType.DMA((2,)),
        ],
    )(week_p, hour_p, minute_p, tw, th, tm)

    xs = pl.pallas_call(
        _tc_xs_body,
        grid=(steps,),
        in_specs=[pl.BlockSpec((N, H), lambda i: (0, 0))],
        out_specs=pl.BlockSpec(memory_space=pl.ANY),
        scratch_shapes=[
            pltpu.VMEM((_G * N, H), jnp.float32),
            pltpu.SemaphoreType.DMA((2,)),
        ],
        out_shape=jax.ShapeDtypeStruct((steps, _G * N, H), jnp.float32),
    )(node_embedding)

    return xs.reshape(B, L, N, H), xt.reshape(B, L, N, H)


# speedup vs baseline: 1.7963x; 1.7963x over previous
import jax
import jax.numpy as jnp
from jax.experimental import pallas as pl
from jax.experimental.pallas import tpu as pltpu

interpret = False

_G = 8


def _body(tw_ref, th_ref, tm_ref, node_ref, week_ref, hour_ref, minute_ref,
          xs_ref, xt_ref, xs_rep, xt_build, sem_xs, sem_xt):
    i = pl.program_id(0)
    nsteps = pl.num_programs(0)
    n, h = node_ref.shape
    slot = jax.lax.rem(i, 3)

    @pl.when(i == 0)
    def _():
        for r in range(_G):
            xs_rep[r * n:(r + 1) * n, :] = node_ref[...]

    @pl.when(i >= 3)
    def _():
        pltpu.make_async_copy(xs_rep, xs_ref.at[i - 3], sem_xs.at[slot]).wait()
    pltpu.make_async_copy(xs_rep, xs_ref.at[i], sem_xs.at[slot]).start()

    @pl.when(i >= 3)
    def _():
        pltpu.make_async_copy(xt_build.at[slot], xt_ref.at[i - 3],
                              sem_xt.at[slot]).wait()
    for g in range(_G):
        idx = i * _G + g
        row = (week_ref[pl.ds(tw_ref[idx], 1), :]
               + hour_ref[pl.ds(th_ref[idx], 1), :]
               + minute_ref[pl.ds(tm_ref[idx], 1), :])
        xt_build[slot, g * n:(g + 1) * n, :] = jnp.broadcast_to(row, (n, h))
    pltpu.make_async_copy(xt_build.at[slot], xt_ref.at[i],
                          sem_xt.at[slot]).start()

    @pl.when(i == nsteps - 1)
    def _():
        for d in range(2, -1, -1):
            s = jax.lax.rem(i - d, 3)
            pltpu.make_async_copy(xt_build.at[s], xt_ref.at[i - d],
                                  sem_xt.at[s]).wait()
            pltpu.make_async_copy(xs_rep, xs_ref.at[i - d],
                                  sem_xs.at[s]).wait()


def kernel(t, node_embedding, week_table, hour_table, minute_table):
    B, L = t.shape[0], t.shape[1]
    N, H = node_embedding.shape
    wn, wd = week_table.shape
    hn, hd = hour_table.shape
    mn, md = minute_table.shape
    steps = (B * L) // _G

    week_p = jnp.zeros((8, H), jnp.float32).at[:wn, :wd].set(week_table)
    hour_p = jnp.zeros((24, H), jnp.float32).at[:hn, wd:wd + hd].set(hour_table)
    minute_p = jnp.zeros((8, H), jnp.float32).at[:mn, wd + hd:].set(minute_table)

    tw = t[:, :, 0, 1].reshape(-1).astype(jnp.int32)
    th = t[:, :, 0, 2].reshape(-1).astype(jnp.int32)
    tm = t[:, :, 0, 3].reshape(-1).astype(jnp.int32)

    grid_spec = pltpu.PrefetchScalarGridSpec(
        num_scalar_prefetch=3,
        grid=(steps,),
        in_specs=[
            pl.BlockSpec((N, H), lambda i, *_: (0, 0)),
            pl.BlockSpec((8, H), lambda i, *_: (0, 0)),
            pl.BlockSpec((24, H), lambda i, *_: (0, 0)),
            pl.BlockSpec((8, H), lambda i, *_: (0, 0)),
        ],
        out_specs=[
            pl.BlockSpec(memory_space=pl.ANY),
            pl.BlockSpec(memory_space=pl.ANY),
        ],
        scratch_shapes=[
            pltpu.VMEM((_G * N, H), jnp.float32),
            pltpu.VMEM((3, _G * N, H), jnp.float32),
            pltpu.SemaphoreType.DMA((3,)),
            pltpu.SemaphoreType.DMA((3,)),
        ],
    )
    xs, xt = pl.pallas_call(
        _body,
        grid_spec=grid_spec,
        out_shape=[jax.ShapeDtypeStruct((steps, _G * N, H), jnp.float32)] * 2,
        interpret=interpret,
    )(tw, th, tm, node_embedding, week_p, hour_p, minute_p)
    return xs.reshape(B, L, N, H), xt.reshape(B, L, N, H)
